# baseline (speedup 1.0000x reference)
import functools

import jax
import jax.numpy as jnp
from jax import lax
from jax.experimental import pallas as pl
from jax.experimental.pallas import tpu as pltpu
from jax.experimental.pallas import tpu_sc as plsc

NC = 2
NS = 16
NW = NC * NS
LANES = 16


def _emb_body(x_hbm, tokT_hbm, posT_hbm, outT_hbm, row_v, idx_v, pos_v, o_v,
              *, B, T, V, E):
    wid = lax.axis_index("s") * NC + lax.axis_index("c")
    e_per_w = E // NW

    for el in range(e_per_w):
        e = wid * e_per_w + el
        pltpu.sync_copy(tokT_hbm.at[e], row_v)
        pltpu.sync_copy(posT_hbm.at[e], pos_v)

        def b_body(b, _):
            pltpu.sync_copy(x_hbm.at[b], idx_v)

            def g_body(g, _):
                idx = idx_v[pl.ds(g * LANES, LANES)]
                vals = plsc.load_gather(row_v, [idx])
                o_v[pl.ds(g * LANES, LANES)] = vals + pos_v[pl.ds(g * LANES, LANES)]
                return 0

            lax.fori_loop(0, T // LANES, g_body, 0, unroll=8)
            pltpu.sync_copy(o_v, outT_hbm.at[b, e])
            return 0

        lax.fori_loop(0, B, b_body, 0)


def kernel(x, token_embedding_table, position_embedding_table):
    B, T = x.shape
    V, E = token_embedding_table.shape
    x32 = x.astype(jnp.int32)
    tokT = token_embedding_table.T
    posT = position_embedding_table.T

    mesh = plsc.VectorSubcoreMesh(core_axis_name="c", subcore_axis_name="s")
    body = functools.partial(_emb_body, B=B, T=T, V=V, E=E)
    run = pl.kernel(
        body,
        out_type=jax.ShapeDtypeStruct((B, E, T), jnp.float32),
        mesh=mesh,
        scratch_types=[
            pltpu.VMEM((V,), jnp.float32),
            pltpu.VMEM((T,), jnp.int32),
            pltpu.VMEM((T,), jnp.float32),
            pltpu.VMEM((T,), jnp.float32),
        ],
        compiler_params=pltpu.CompilerParams(needs_layout_passes=False),
    )
    outT = run(x32, tokT, posT)
    return outT.transpose(0, 2, 1)
